# baseline (speedup 1.0000x reference)
import functools

import jax
import jax.numpy as jnp
from jax import lax
from jax.experimental import pallas as pl
from jax.experimental.pallas import tpu as pltpu
from jax.experimental.pallas import tpu_sc as plsc

_N = 10000
_E = 320000
_D = 128
_R = 8
_NR = _N * _R
_CW = 16
_NC = 2
_NS = 16
_NW = _NC * _NS
_EPW = _E // _NW
_C = 80
_NCH = _EPW // _C
_RPT = _N // _NS
_CRPT = _NR // _NS
_IROWS = _NR // _D


def _lane_iota():
    return lax.broadcasted_iota(jnp.int32, (16,), 0)


def _count_body(dst_hbm, et_hbm, out_hbm,
                dfull, efull, seg0, seg1, ones_v, zb_v, cnt_sh, sem0, sem1):
    c = lax.axis_index("c")
    s = lax.axis_index("s")
    wid = s * _NC + c
    segb = (seg0, seg1)
    semb = (sem0, sem1)

    def zb(i, _):
        zb_v[i, pl.ds(0, 16)] = jnp.zeros((16,), jnp.float32)
        return 0
    lax.fori_loop(0, 1000, zb, 0)
    for k in range(_CRPT // 1000):
        pltpu.sync_copy(zb_v, cnt_sh.at[pl.ds(s * _CRPT + k * 1000, 1000)])

    one_row = jnp.where(_lane_iota() == 0, 1.0, 0.0).astype(jnp.float32)
    def ob(i, _):
        ones_v[i, pl.ds(0, 16)] = one_row
        return 0
    lax.fori_loop(0, _C, ob, 0)

    pltpu.sync_copy(dst_hbm.at[pl.ds(wid * _EPW, _EPW)], dfull)
    pltpu.sync_copy(et_hbm.at[pl.ds(wid * _EPW, _EPW)], efull)

    plsc.subcore_barrier()

    def fire(g, b):
        off = g * _C
        for q in range(_C // 16):
            d16 = dfull[pl.ds(off + q * 16, 16)]
            e16 = efull[pl.ds(off + q * 16, 16)]
            segb[b][pl.ds(q * 16, 16)] = d16 * _R + e16
        pltpu.async_copy(ones_v, cnt_sh.at[segb[b]], semb[b], add=True)

    def drain(b):
        pltpu.make_async_copy(ones_v, cnt_sh.at[segb[b]], semb[b]).wait()

    fire(0, 0)
    def outer(it, _):
        fire(2 * it + 1, 1)
        drain(0)
        fire(2 * it + 2, 0)
        drain(1)
        return 0
    lax.fori_loop(0, (_NCH - 1) // 2, outer, 0)
    drain(0)

    plsc.subcore_barrier()

    for k in range(_CRPT // 1000):
        off = s * _CRPT + k * 1000
        pltpu.sync_copy(cnt_sh.at[pl.ds(off, 1000)], zb_v)
        pltpu.sync_copy(zb_v, out_hbm.at[c].at[pl.ds(off, 1000)])


def _sc_counts(dst, et):
    mesh = plsc.VectorSubcoreMesh(core_axis_name="c", subcore_axis_name="s",
                                  num_cores=_NC, num_subcores=_NS)
    return pl.kernel(
        _count_body,
        out_type=jax.ShapeDtypeStruct((_NC, _NR, _CW), jnp.float32),
        mesh=mesh,
        compiler_params=pltpu.CompilerParams(use_tc_tiling_on_sc=False),
        scratch_types=[
            pltpu.VMEM((_EPW,), jnp.int32),
            pltpu.VMEM((_EPW,), jnp.int32),
            pltpu.VMEM((_C,), jnp.int32),
            pltpu.VMEM((_C,), jnp.int32),
            pltpu.VMEM((_C, _CW), jnp.float32),
            pltpu.VMEM((1000, _CW), jnp.float32),
            pltpu.VMEM_SHARED((_NR, _CW), jnp.float32),
            pltpu.SemaphoreType.DMA,
            pltpu.SemaphoreType.DMA,
        ],
    )(dst, et)


_BN = 2000


def _ytab1_body(x_ref, w_ref, cnt_ref, y_ref, inv_ref):
    r = pl.program_id(0)
    i = pl.program_id(1)
    @pl.when(jnp.logical_and(r == 0, i == 0))
    def _():
        total = cnt_ref[0] + cnt_ref[1]
        inv_ref[...] = 1.0 / jnp.clip(total, 1.0, None)
    y_ref[0] = jnp.dot(x_ref[...], w_ref[0],
                       preferred_element_type=jnp.float32)


def _tc_ytable1(x, Wrel, cnt2d):
    grid = (_R, _N // _BN)
    return pl.pallas_call(
        _ytab1_body,
        grid=grid,
        in_specs=[
            pl.BlockSpec((_BN, _D), lambda r, i: (i, 0)),
            pl.BlockSpec((1, _D, _D), lambda r, i: (r, 0, 0)),
            pl.BlockSpec((_NC, _IROWS, _D), lambda r, i: (0, 0, 0)),
        ],
        out_specs=[
            pl.BlockSpec((1, _BN, _D), lambda r, i: (r, i, 0)),
            pl.BlockSpec((_IROWS, _D), lambda r, i: (0, 0)),
        ],
        out_shape=[
            jax.ShapeDtypeStruct((_R, _N, _D), jnp.float32),
            jax.ShapeDtypeStruct((_IROWS, _D), jnp.float32),
        ],
    )(x, Wrel, cnt2d)


def _comb1_ytab2_body(x_ref, w_ref, b_ref, p_ref, w2_ref, h_ref, y_ref):
    t = jnp.dot(x_ref[...], w_ref[...], preferred_element_type=jnp.float32)
    h = jnp.maximum(t + b_ref[0] + p_ref[0] + p_ref[1], 0.0)
    h_ref[...] = h
    for r in range(_R):
        y_ref[r] = jnp.dot(h, w2_ref[r], preferred_element_type=jnp.float32)


def _tc_comb1_ytab2(x, Wroot, b, partials, Wrel2):
    grid = (_N // _BN,)
    return pl.pallas_call(
        _comb1_ytab2_body,
        grid=grid,
        in_specs=[
            pl.BlockSpec((_BN, _D), lambda i: (i, 0)),
            pl.BlockSpec((_D, _D), lambda i: (0, 0)),
            pl.BlockSpec((1, _D), lambda i: (0, 0)),
            pl.BlockSpec((_NC, _BN, _D), lambda i: (0, i, 0)),
            pl.BlockSpec((_R, _D, _D), lambda i: (0, 0, 0)),
        ],
        out_specs=[
            pl.BlockSpec((_BN, _D), lambda i: (i, 0)),
            pl.BlockSpec((_R, _BN, _D), lambda i: (0, i, 0)),
        ],
        out_shape=[
            jax.ShapeDtypeStruct((_N, _D), jnp.float32),
            jax.ShapeDtypeStruct((_R, _N, _D), jnp.float32),
        ],
    )(x, Wroot, b.reshape(1, _D), partials, Wrel2)


def _agg_body(y_hbm, src_hbm, dst_hbm, et_hbm, inv_hbm, out_hbm,
              gfull, efull, dfull, rows0, rows1, inv0, inv1,
              acc_sh, semr0, semr1, semi0, semi1):
    c = lax.axis_index("c")
    s = lax.axis_index("s")
    wid = s * _NC + c
    rowsb = (rows0, rows1)
    invb = (inv0, inv1)
    semrb = (semr0, semr1)
    semib = (semi0, semi1)

    def zb(i, _):
        for j in range(_D // 16):
            rows0[i, pl.ds(j * 16, 16)] = jnp.zeros((16,), jnp.float32)
        return 0
    lax.fori_loop(0, _C, zb, 0)
    for k in range(_RPT // _C):
        pltpu.sync_copy(rows0, acc_sh.at[pl.ds(s * _RPT + k * _C, _C)])
    pltpu.sync_copy(rows0.at[pl.ds(0, _RPT % _C)],
                    acc_sh.at[pl.ds(s * _RPT + (_RPT // _C) * _C,
                                    _RPT % _C)])

    pltpu.sync_copy(src_hbm.at[pl.ds(wid * _EPW, _EPW)], gfull)
    pltpu.sync_copy(et_hbm.at[pl.ds(wid * _EPW, _EPW)], efull)
    pltpu.sync_copy(dst_hbm.at[wid], dfull)

    def pidx(i, _):
        g = i // (_C // 16)
        q = i % (_C // 16)
        s16 = gfull[pl.ds(i * 16, 16)]
        e16 = efull[pl.ds(i * 16, 16)]
        d16 = dfull[g, pl.ds(q * 16, 16)]
        gfull[pl.ds(i * 16, 16)] = e16 * _N + s16
        efull[pl.ds(i * 16, 16)] = d16 * _R + e16
        return 0
    lax.fori_loop(0, _EPW // 16, pidx, 0)

    plsc.subcore_barrier()

    def fire(g, b):
        off = g * _C
        pltpu.async_copy(y_hbm.at[gfull.at[pl.ds(off, _C)]], rowsb[b],
                         semrb[b])
        pltpu.async_copy(inv_hbm.at[efull.at[pl.ds(off, _C)]], invb[b],
                         semib[b])

    def finish(g, b):
        off = g * _C
        pltpu.make_async_copy(y_hbm.at[gfull.at[pl.ds(off, _C)]], rowsb[b],
                              semrb[b]).wait()
        pltpu.make_async_copy(inv_hbm.at[efull.at[pl.ds(off, _C)]], invb[b],
                              semib[b]).wait()
        for q in range(_C // 16):
            inv16 = invb[b][pl.ds(q * 16, 16)]
            for i in range(16):
                fv = jnp.full((16,), inv16[i], jnp.float32)
                e = q * 16 + i
                for j in range(_D // 16):
                    rowsb[b][e, pl.ds(j * 16, 16)] = (
                        rowsb[b][e, pl.ds(j * 16, 16)] * fv)
        pltpu.sync_copy(rowsb[b], acc_sh.at[dfull.at[g]], add=True)

    fire(0, 0)
    def outer(it, _):
        fire(2 * it + 1, 1)
        finish(2 * it, 0)
        fire(2 * it + 2, 0)
        finish(2 * it + 1, 1)
        return 0
    lax.fori_loop(0, (_NCH - 1) // 2, outer, 0)
    finish(_NCH - 1, 0)

    plsc.subcore_barrier()

    for k in range(_RPT // _C):
        off = s * _RPT + k * _C
        pltpu.sync_copy(acc_sh.at[pl.ds(off, _C)], rows0)
        pltpu.sync_copy(rows0, out_hbm.at[c].at[pl.ds(off, _C)])
    off = s * _RPT + (_RPT // _C) * _C
    pltpu.sync_copy(acc_sh.at[pl.ds(off, _RPT % _C)],
                    rows0.at[pl.ds(0, _RPT % _C)])
    pltpu.sync_copy(rows0.at[pl.ds(0, _RPT % _C)],
                    out_hbm.at[c].at[pl.ds(off, _RPT % _C)])


def _sc_aggregate(y, src, dst, et, invf):
    mesh = plsc.VectorSubcoreMesh(core_axis_name="c", subcore_axis_name="s",
                                  num_cores=_NC, num_subcores=_NS)
    return pl.kernel(
        _agg_body,
        out_type=jax.ShapeDtypeStruct((_NC, _N, _D), jnp.float32),
        mesh=mesh,
        compiler_params=pltpu.CompilerParams(use_tc_tiling_on_sc=False),
        scratch_types=[
            pltpu.VMEM((_EPW,), jnp.int32),
            pltpu.VMEM((_EPW,), jnp.int32),
            pltpu.VMEM((_NCH, _C), jnp.int32),
            pltpu.VMEM((_C, _D), jnp.float32),
            pltpu.VMEM((_C, _D), jnp.float32),
            pltpu.VMEM((_C,), jnp.float32),
            pltpu.VMEM((_C,), jnp.float32),
            pltpu.VMEM_SHARED((_N, _D), jnp.float32),
            pltpu.SemaphoreType.DMA,
            pltpu.SemaphoreType.DMA,
            pltpu.SemaphoreType.DMA,
            pltpu.SemaphoreType.DMA,
        ],
    )(y, src, dst.reshape(_NW, _NCH, _C), et, invf)


def _comb1_body(x_ref, w_ref, b_ref, p_ref, h_ref):
    t = jnp.dot(x_ref[...], w_ref[...], preferred_element_type=jnp.float32)
    h_ref[...] = jnp.maximum(t + b_ref[0] + p_ref[0] + p_ref[1], 0.0)


def _tc_combine1(x, Wroot, b, partials):
    grid = (_N // _BN,)
    return pl.pallas_call(
        _comb1_body,
        grid=grid,
        in_specs=[
            pl.BlockSpec((_BN, _D), lambda i: (i, 0)),
            pl.BlockSpec((_D, _D), lambda i: (0, 0)),
            pl.BlockSpec((1, _D), lambda i: (0, 0)),
            pl.BlockSpec((_NC, _BN, _D), lambda i: (0, i, 0)),
        ],
        out_specs=pl.BlockSpec((_BN, _D), lambda i: (i, 0)),
        out_shape=jax.ShapeDtypeStruct((_N, _D), jnp.float32),
    )(x, Wroot, b.reshape(1, _D), partials)


def _comb2_body(h_ref, w_ref, b_ref, p_ref, wl_ref, bl_ref, o_ref):
    t = jnp.dot(h_ref[...], w_ref[...], preferred_element_type=jnp.float32)
    t = t + b_ref[0] + p_ref[0] + p_ref[1]
    o_ref[...] = jnp.dot(t, wl_ref[...],
                         preferred_element_type=jnp.float32) + bl_ref[0]


def _tc_combine2(h, Wroot, b, partials, Wlin, blin):
    grid = (_N // _BN,)
    return pl.pallas_call(
        _comb2_body,
        grid=grid,
        in_specs=[
            pl.BlockSpec((_BN, _D), lambda i: (i, 0)),
            pl.BlockSpec((_D, _D), lambda i: (0, 0)),
            pl.BlockSpec((1, _D), lambda i: (0, 0)),
            pl.BlockSpec((_NC, _BN, _D), lambda i: (0, i, 0)),
            pl.BlockSpec((_D, _D), lambda i: (0, 0)),
            pl.BlockSpec((1, _D), lambda i: (0, 0)),
        ],
        out_specs=pl.BlockSpec((_BN, _D), lambda i: (i, 0)),
        out_shape=jax.ShapeDtypeStruct((_N, _D), jnp.float32),
    )(h, Wroot, b.reshape(1, _D), partials, Wlin, blin.reshape(1, _D))


def kernel(x, ei, et, Wrel1, Wroot1, b1, Wrel2, Wroot2, b2, Wlin, blin):
    src = ei[0]
    dst = ei[1]

    cnt = _sc_counts(dst, et)
    cnt2d = cnt[:, :, 0].reshape(_NC, _IROWS, _D)
    y1, inv2d = _tc_ytable1(x, Wrel1, cnt2d)
    invf = inv2d.reshape(_NR)
    p1 = _sc_aggregate(y1.reshape(_R * _N, _D), src, dst, et, invf)
    h, y2 = _tc_comb1_ytab2(x, Wroot1, b1, p1, Wrel2)
    p2 = _sc_aggregate(y2.reshape(_R * _N, _D), src, dst, et, invf)
    return _tc_combine2(h, Wroot2, b2, p2, Wlin, blin)


# speedup vs baseline: 1.0075x; 1.0075x over previous
import jax
import jax.numpy as jnp
from jax import lax
from jax.experimental import pallas as pl
from jax.experimental.pallas import tpu as pltpu
from jax.experimental.pallas import tpu_sc as plsc

_N = 10000
_E = 320000
_D = 128
_R = 8
_NR = _N * _R
_CW = 16
_NC = 2
_NS = 16
_NW = _NC * _NS
_EPW = _E // _NW
_C = 80
_NCH = _EPW // _C
_RPT = _N // _NS
_CRPT = _NR // _NS
_IROWS = _NR // _D


def _lane_iota():
    return lax.broadcasted_iota(jnp.int32, (16,), 0)


def _count_body(dst_hbm, et_hbm, out_hbm,
                dfull, efull, seg0, seg1, ones_v, zb_v, cnt_sh, sem0, sem1):
    c = lax.axis_index("c")
    s = lax.axis_index("s")
    wid = s * _NC + c
    segb = (seg0, seg1)
    semb = (sem0, sem1)

    def zb(i, _):
        zb_v[i, pl.ds(0, 16)] = jnp.zeros((16,), jnp.float32)
        return 0
    lax.fori_loop(0, 1000, zb, 0)
    for k in range(_CRPT // 1000):
        pltpu.sync_copy(zb_v, cnt_sh.at[pl.ds(s * _CRPT + k * 1000, 1000)])

    one_row = jnp.where(_lane_iota() == 0, 1.0, 0.0).astype(jnp.float32)
    def ob(i, _):
        ones_v[i, pl.ds(0, 16)] = one_row
        return 0
    lax.fori_loop(0, _C, ob, 0)

    pltpu.sync_copy(dst_hbm.at[pl.ds(wid * _EPW, _EPW)], dfull)
    pltpu.sync_copy(et_hbm.at[pl.ds(wid * _EPW, _EPW)], efull)

    plsc.subcore_barrier()

    def fire(g, b):
        off = g * _C
        for q in range(_C // 16):
            d16 = dfull[pl.ds(off + q * 16, 16)]
            e16 = efull[pl.ds(off + q * 16, 16)]
            segb[b][pl.ds(q * 16, 16)] = d16 * _R + e16
        pltpu.async_copy(ones_v, cnt_sh.at[segb[b]], semb[b], add=True)

    def drain(b):
        pltpu.make_async_copy(ones_v, cnt_sh.at[segb[b]], semb[b]).wait()

    fire(0, 0)
    def outer(it, _):
        fire(2 * it + 1, 1)
        drain(0)
        fire(2 * it + 2, 0)
        drain(1)
        return 0
    lax.fori_loop(0, (_NCH - 1) // 2, outer, 0)
    drain(0)

    plsc.subcore_barrier()

    for k in range(_CRPT // 1000):
        off = s * _CRPT + k * 1000
        pltpu.sync_copy(cnt_sh.at[pl.ds(off, 1000)], zb_v)
        pltpu.sync_copy(zb_v, out_hbm.at[c].at[pl.ds(off, 1000)])


def _sc_counts(dst, et):
    mesh = plsc.VectorSubcoreMesh(core_axis_name="c", subcore_axis_name="s",
                                  num_cores=_NC, num_subcores=_NS)
    return pl.kernel(
        _count_body,
        out_type=jax.ShapeDtypeStruct((_NC, _NR, _CW), jnp.float32),
        mesh=mesh,
        compiler_params=pltpu.CompilerParams(use_tc_tiling_on_sc=False),
        scratch_types=[
            pltpu.VMEM((_EPW,), jnp.int32),
            pltpu.VMEM((_EPW,), jnp.int32),
            pltpu.VMEM((_C,), jnp.int32),
            pltpu.VMEM((_C,), jnp.int32),
            pltpu.VMEM((_C, _CW), jnp.float32),
            pltpu.VMEM((1000, _CW), jnp.float32),
            pltpu.VMEM_SHARED((_NR, _CW), jnp.float32),
            pltpu.SemaphoreType.DMA,
            pltpu.SemaphoreType.DMA,
        ],
    )(dst, et)


_BN = 2000


def _ytab1_body(x_ref, w_ref, cnt_ref, y_ref, inv_ref):
    r = pl.program_id(0)
    i = pl.program_id(1)
    @pl.when(jnp.logical_and(r == 0, i == 0))
    def _():
        total = cnt_ref[0] + cnt_ref[1]
        inv_ref[...] = 1.0 / jnp.clip(total, 1.0, None)
    y_ref[0] = jnp.dot(x_ref[...], w_ref[0],
                       preferred_element_type=jnp.float32)


def _tc_ytable1(x, Wrel, cnt2d):
    grid = (_R, _N // _BN)
    return pl.pallas_call(
        _ytab1_body,
        grid=grid,
        in_specs=[
            pl.BlockSpec((_BN, _D), lambda r, i: (i, 0)),
            pl.BlockSpec((1, _D, _D), lambda r, i: (r, 0, 0)),
            pl.BlockSpec((_NC, _IROWS, _D), lambda r, i: (0, 0, 0)),
        ],
        out_specs=[
            pl.BlockSpec((1, _BN, _D), lambda r, i: (r, i, 0)),
            pl.BlockSpec((_IROWS, _D), lambda r, i: (0, 0)),
        ],
        out_shape=[
            jax.ShapeDtypeStruct((_R, _N, _D), jnp.float32),
            jax.ShapeDtypeStruct((_IROWS, _D), jnp.float32),
        ],
    )(x, Wrel, cnt2d)


def _comb1_ytab2_body(x_ref, w_ref, b_ref, p_ref, w2_ref, h_ref, y_ref):
    t = jnp.dot(x_ref[...], w_ref[...], preferred_element_type=jnp.float32)
    h = jnp.maximum(t + b_ref[0] + p_ref[0] + p_ref[1], 0.0)
    h_ref[...] = h
    for r in range(_R):
        y_ref[r] = jnp.dot(h, w2_ref[r], preferred_element_type=jnp.float32)


def _tc_comb1_ytab2(x, Wroot, b, partials, Wrel2):
    grid = (_N // _BN,)
    return pl.pallas_call(
        _comb1_ytab2_body,
        grid=grid,
        in_specs=[
            pl.BlockSpec((_BN, _D), lambda i: (i, 0)),
            pl.BlockSpec((_D, _D), lambda i: (0, 0)),
            pl.BlockSpec((1, _D), lambda i: (0, 0)),
            pl.BlockSpec((_NC, _BN, _D), lambda i: (0, i, 0)),
            pl.BlockSpec((_R, _D, _D), lambda i: (0, 0, 0)),
        ],
        out_specs=[
            pl.BlockSpec((_BN, _D), lambda i: (i, 0)),
            pl.BlockSpec((_R, _BN, _D), lambda i: (0, i, 0)),
        ],
        out_shape=[
            jax.ShapeDtypeStruct((_N, _D), jnp.float32),
            jax.ShapeDtypeStruct((_R, _N, _D), jnp.float32),
        ],
    )(x, Wroot, b.reshape(1, _D), partials, Wrel2)


def _agg_body(y_hbm, src_hbm, dst_hbm, et_hbm, inv_hbm, out_hbm,
              gfull, efull, dfull, rows0, rows1, inv0, inv1, dt0, dt1,
              acc_sh, semr0, semr1, semi0, semi1):
    c = lax.axis_index("c")
    s = lax.axis_index("s")
    wid = s * _NC + c
    rowsb = (rows0, rows1)
    invb = (inv0, inv1)
    dtb = (dt0, dt1)
    semrb = (semr0, semr1)
    semib = (semi0, semi1)

    def zb(i, _):
        for j in range(_D // 16):
            rows0[i, pl.ds(j * 16, 16)] = jnp.zeros((16,), jnp.float32)
        return 0
    lax.fori_loop(0, _C, zb, 0)
    for k in range(_RPT // _C):
        pltpu.sync_copy(rows0, acc_sh.at[pl.ds(s * _RPT + k * _C, _C)])
    pltpu.sync_copy(rows0.at[pl.ds(0, _RPT % _C)],
                    acc_sh.at[pl.ds(s * _RPT + (_RPT // _C) * _C,
                                    _RPT % _C)])

    pltpu.sync_copy(src_hbm.at[pl.ds(wid * _EPW, _EPW)], gfull)
    pltpu.sync_copy(et_hbm.at[pl.ds(wid * _EPW, _EPW)], efull)
    pltpu.sync_copy(dst_hbm.at[pl.ds(wid * _EPW, _EPW)], dfull)

    def pidx(i, _):
        s16 = gfull[pl.ds(i * 16, 16)]
        e16 = efull[pl.ds(i * 16, 16)]
        d16 = dfull[pl.ds(i * 16, 16)]
        gfull[pl.ds(i * 16, 16)] = e16 * _N + s16
        efull[pl.ds(i * 16, 16)] = d16 * _R + e16
        return 0
    lax.fori_loop(0, _EPW // 16, pidx, 0)

    plsc.subcore_barrier()

    def fire(g, b):
        off = g * _C
        for q in range(_C // 16):
            dtb[b][pl.ds(q * 16, 16)] = dfull[pl.ds(off + q * 16, 16)]
        pltpu.async_copy(y_hbm.at[gfull.at[pl.ds(off, _C)]], rowsb[b],
                         semrb[b])
        pltpu.async_copy(inv_hbm.at[efull.at[pl.ds(off, _C)]], invb[b],
                         semib[b])

    def finish(g, b):
        off = g * _C
        pltpu.make_async_copy(y_hbm.at[gfull.at[pl.ds(off, _C)]], rowsb[b],
                              semrb[b]).wait()
        pltpu.make_async_copy(inv_hbm.at[efull.at[pl.ds(off, _C)]], invb[b],
                              semib[b]).wait()
        for q in range(_C // 16):
            inv16 = invb[b][pl.ds(q * 16, 16)]
            for i in range(16):
                fv = jnp.full((16,), inv16[i], jnp.float32)
                e = q * 16 + i
                for j in range(_D // 16):
                    rowsb[b][e, pl.ds(j * 16, 16)] = (
                        rowsb[b][e, pl.ds(j * 16, 16)] * fv)
        pltpu.sync_copy(rowsb[b], acc_sh.at[dtb[b]], add=True)

    fire(0, 0)
    def outer(it, _):
        fire(2 * it + 1, 1)
        finish(2 * it, 0)
        fire(2 * it + 2, 0)
        finish(2 * it + 1, 1)
        return 0
    lax.fori_loop(0, (_NCH - 1) // 2, outer, 0)
    finish(_NCH - 1, 0)

    plsc.subcore_barrier()

    for k in range(_RPT // _C):
        off = s * _RPT + k * _C
        pltpu.sync_copy(acc_sh.at[pl.ds(off, _C)], rows0)
        pltpu.sync_copy(rows0, out_hbm.at[c].at[pl.ds(off, _C)])
    off = s * _RPT + (_RPT // _C) * _C
    pltpu.sync_copy(acc_sh.at[pl.ds(off, _RPT % _C)],
                    rows0.at[pl.ds(0, _RPT % _C)])
    pltpu.sync_copy(rows0.at[pl.ds(0, _RPT % _C)],
                    out_hbm.at[c].at[pl.ds(off, _RPT % _C)])


def _sc_aggregate(y, src, dst, et, invf):
    mesh = plsc.VectorSubcoreMesh(core_axis_name="c", subcore_axis_name="s",
                                  num_cores=_NC, num_subcores=_NS)
    return pl.kernel(
        _agg_body,
        out_type=jax.ShapeDtypeStruct((_NC, _N, _D), jnp.float32),
        mesh=mesh,
        compiler_params=pltpu.CompilerParams(use_tc_tiling_on_sc=False),
        scratch_types=[
            pltpu.VMEM((_EPW,), jnp.int32),
            pltpu.VMEM((_EPW,), jnp.int32),
            pltpu.VMEM((_EPW,), jnp.int32),
            pltpu.VMEM((_C, _D), jnp.float32),
            pltpu.VMEM((_C, _D), jnp.float32),
            pltpu.VMEM((_C,), jnp.float32),
            pltpu.VMEM((_C,), jnp.float32),
            pltpu.VMEM((_C,), jnp.int32),
            pltpu.VMEM((_C,), jnp.int32),
            pltpu.VMEM_SHARED((_N, _D), jnp.float32),
            pltpu.SemaphoreType.DMA,
            pltpu.SemaphoreType.DMA,
            pltpu.SemaphoreType.DMA,
            pltpu.SemaphoreType.DMA,
        ],
    )(y, src, dst, et, invf)


def _comb2_body(h_ref, w_ref, b_ref, p_ref, wl_ref, bl_ref, o_ref):
    t = jnp.dot(h_ref[...], w_ref[...], preferred_element_type=jnp.float32)
    t = t + b_ref[0] + p_ref[0] + p_ref[1]
    o_ref[...] = jnp.dot(t, wl_ref[...],
                         preferred_element_type=jnp.float32) + bl_ref[0]


def _tc_combine2(h, Wroot, b, partials, Wlin, blin):
    grid = (_N // _BN,)
    return pl.pallas_call(
        _comb2_body,
        grid=grid,
        in_specs=[
            pl.BlockSpec((_BN, _D), lambda i: (i, 0)),
            pl.BlockSpec((_D, _D), lambda i: (0, 0)),
            pl.BlockSpec((1, _D), lambda i: (0, 0)),
            pl.BlockSpec((_NC, _BN, _D), lambda i: (0, i, 0)),
            pl.BlockSpec((_D, _D), lambda i: (0, 0)),
            pl.BlockSpec((1, _D), lambda i: (0, 0)),
        ],
        out_specs=pl.BlockSpec((_BN, _D), lambda i: (i, 0)),
        out_shape=jax.ShapeDtypeStruct((_N, _D), jnp.float32),
    )(h, Wroot, b.reshape(1, _D), partials, Wlin, blin.reshape(1, _D))


def kernel(x, ei, et, Wrel1, Wroot1, b1, Wrel2, Wroot2, b2, Wlin, blin):
    src = ei[0]
    dst = ei[1]

    cnt = _sc_counts(dst, et)
    cnt2d = cnt[:, :, 0].reshape(_NC, _IROWS, _D)

    y1, inv2d = _tc_ytable1(x, Wrel1, cnt2d)
    invf = inv2d.reshape(_NR)
    p1 = _sc_aggregate(y1.reshape(_R * _N, _D), src, dst, et, invf)
    h, y2 = _tc_comb1_ytab2(x, Wroot1, b1, p1, Wrel2)
    p2 = _sc_aggregate(y2.reshape(_R * _N, _D), src, dst, et, invf)
    return _tc_combine2(h, Wroot2, b2, p2, Wlin, blin)
